# baseline (speedup 1.0000x reference)
import functools

import jax
import jax.numpy as jnp
from jax import lax
from jax.experimental import pallas as pl
from jax.experimental.pallas import tpu as pltpu
from jax.experimental.pallas import tpu_sc as plsc

MAX_POS = 8192
HIDDEN = 768
B = 4
L = 8192

N_TOK = B * L
LANES = 16
NC = 2
NS = 16
NW = NC * NS
TPW = N_TOK // NW
CHUNK = 16
NGROUP = TPW // CHUNK
NVREG = HIDDEN // LANES

_EPS = 1e-12
_INV_H = 1.0 / HIDDEN


def _rsqrt(u):
    yi = lax.bitcast_convert_type(u, jnp.int32)
    yi = jnp.full((LANES,), 0x5F3759DF, jnp.int32) - lax.shift_right_logical(
        yi, jnp.full((LANES,), 1, jnp.int32))
    g = lax.bitcast_convert_type(yi, jnp.float32)
    for _ in range(3):
        g = g * (1.5 - 0.5 * u * g * g)
    return g


_GATHER_DNUMS = lax.GatherDimensionNumbers(
    offset_dims=(), collapsed_slice_dims=(0,), start_index_map=(0,))


def _lane_sum(v):
    lanes = lax.iota(jnp.int32, LANES)
    for k in (8, 4, 2, 1):
        idx = lax.reshape(lanes ^ k, (LANES, 1))
        v = v + lax.gather(v, idx, _GATHER_DNUMS, (1,),
                           mode=lax.GatherScatterMode.PROMISE_IN_BOUNDS)
    return v


def _body(x_hbm, idx_hbm, t1_hbm, t2_hbm, w_hbm, b_hbm, out_hbm,
          idx_v, r1_v, r2_v, x_v, o_v, w_v, b_v, sem):
    wid = lax.axis_index("s") * NC + lax.axis_index("c")
    base0 = wid * TPW
    pltpu.sync_copy(w_hbm, w_v)
    pltpu.sync_copy(b_hbm, b_v)

    def group(g, carry):
        base = base0 + g * CHUNK
        pltpu.sync_copy(idx_hbm.at[pl.ds(base, CHUNK)], idx_v)
        pltpu.async_copy(t1_hbm.at[idx_v], r1_v, sem)
        pltpu.async_copy(t2_hbm.at[idx_v], r2_v, sem)
        pltpu.async_copy(x_hbm.at[pl.ds(base, CHUNK)], x_v, sem)
        pltpu.make_async_copy(t1_hbm.at[idx_v], r1_v, sem).wait()
        pltpu.make_async_copy(t2_hbm.at[idx_v], r2_v, sem).wait()
        pltpu.make_async_copy(x_hbm.at[pl.ds(base, CHUNK)], x_v, sem).wait()

        def token(t, tc):
            def pass1(j, c):
                s, q = c
                sl = pl.ds(j * LANES, LANES)
                v = r1_v[t, sl] * x_v[t, sl] + r2_v[t, sl]
                o_v[t, sl] = v
                return s + v, q + v * v

            zero = jnp.zeros((LANES,), jnp.float32)
            s, q = lax.fori_loop(0, NVREG, pass1, (zero, zero))
            mean = _lane_sum(s) * _INV_H
            var = _lane_sum(q) * _INV_H - mean * mean
            inv = _rsqrt(var + _EPS)

            def pass2(j, c):
                sl = pl.ds(j * LANES, LANES)
                sv = w_v[sl] * inv
                o_v[t, sl] = (o_v[t, sl] - mean) * sv + b_v[sl]
                return c

            lax.fori_loop(0, NVREG, pass2, 0)
            return tc

        lax.fori_loop(0, CHUNK, token, 0)
        pltpu.sync_copy(o_v, out_hbm.at[pl.ds(base, CHUNK)])
        return carry

    lax.fori_loop(0, NGROUP, group, 0)


@jax.jit
def _fc_embed(x2d, ids, t1, t2, w, b):
    mesh = plsc.VectorSubcoreMesh(core_axis_name="c", subcore_axis_name="s")
    f = functools.partial(
        pl.kernel,
        mesh=mesh,
        out_type=jax.ShapeDtypeStruct((N_TOK, HIDDEN), jnp.float32),
        scratch_types=[
            pltpu.VMEM((CHUNK,), jnp.int32),
            pltpu.VMEM((CHUNK, HIDDEN), jnp.float32),
            pltpu.VMEM((CHUNK, HIDDEN), jnp.float32),
            pltpu.VMEM((CHUNK, HIDDEN), jnp.float32),
            pltpu.VMEM((CHUNK, HIDDEN), jnp.float32),
            pltpu.VMEM((HIDDEN,), jnp.float32),
            pltpu.VMEM((HIDDEN,), jnp.float32),
            pltpu.SemaphoreType.DMA,
        ],
    )(_body)
    return f(x2d, ids, t1, t2, w, b)


def kernel(inputs_embeds, position_ids, pos_table1, pos_table2, ln_weight, ln_bias):
    x2d = inputs_embeds.reshape(N_TOK, HIDDEN)
    ids = position_ids.reshape(N_TOK).astype(jnp.int32)
    out = _fc_embed(x2d, ids, pos_table1, pos_table2, ln_weight, ln_bias)
    return out.reshape(B, L, HIDDEN)


# speedup vs baseline: 1.0966x; 1.0966x over previous
import functools

import jax
import jax.numpy as jnp
from jax import lax
from jax.experimental import pallas as pl
from jax.experimental.pallas import tpu as pltpu
from jax.experimental.pallas import tpu_sc as plsc

MAX_POS = 8192
HIDDEN = 768
B = 4
L = 8192

N_TOK = B * L
LANES = 16
NC = 2
NS = 16
NW = NC * NS
TPW = N_TOK // NW
CHUNK = 16
NGROUP = TPW // CHUNK
NVREG = HIDDEN // LANES

_EPS = 1e-12
_INV_H = 1.0 / HIDDEN


def _rsqrt(u):
    yi = lax.bitcast_convert_type(u, jnp.int32)
    yi = jnp.full((LANES,), 0x5F3759DF, jnp.int32) - lax.shift_right_logical(
        yi, jnp.full((LANES,), 1, jnp.int32))
    g = lax.bitcast_convert_type(yi, jnp.float32)
    for _ in range(3):
        g = g * (1.5 - 0.5 * u * g * g)
    return g


_GATHER_DNUMS = lax.GatherDimensionNumbers(
    offset_dims=(), collapsed_slice_dims=(0,), start_index_map=(0,))


def _lane_sum(v):
    lanes = lax.iota(jnp.int32, LANES)
    for k in (8, 4, 2, 1):
        idx = lax.reshape(lanes ^ k, (LANES, 1))
        v = v + lax.gather(v, idx, _GATHER_DNUMS, (1,),
                           mode=lax.GatherScatterMode.PROMISE_IN_BOUNDS)
    return v


def _body(x_hbm, idx_hbm, t1_hbm, t2_hbm, w_hbm, b_hbm, out_hbm,
          idx_v, r1_v, r2_v, x_v, o_v, w_v, b_v, gsem0, gsem1, osem0, osem1):
    wid = lax.axis_index("s") * NC + lax.axis_index("c")
    base0 = wid * TPW
    gsem = (gsem0, gsem1)
    osem = (osem0, osem1)
    pltpu.sync_copy(w_hbm, w_v)
    pltpu.sync_copy(b_hbm, b_v)

    def fetch(g, b):
        base = base0 + g * CHUNK
        pltpu.sync_copy(idx_hbm.at[pl.ds(base, CHUNK)], idx_v.at[b])
        pltpu.async_copy(t1_hbm.at[idx_v.at[b]], r1_v.at[b], gsem[b])
        pltpu.async_copy(t2_hbm.at[idx_v.at[b]], r2_v.at[b], gsem[b])
        pltpu.async_copy(x_hbm.at[pl.ds(base, CHUNK)], x_v.at[b], gsem[b])

    def wait_fetch(g, b):
        base = base0 + g * CHUNK
        pltpu.make_async_copy(t1_hbm.at[idx_v.at[b]], r1_v.at[b], gsem[b]).wait()
        pltpu.make_async_copy(t2_hbm.at[idx_v.at[b]], r2_v.at[b], gsem[b]).wait()
        pltpu.make_async_copy(x_hbm.at[pl.ds(base, CHUNK)], x_v.at[b], gsem[b]).wait()

    def compute(b):
        def token(t, tc):
            def pass1(j, c):
                s, q = c
                sl = pl.ds(j * LANES, LANES)
                v = r1_v[b, t, sl] * x_v[b, t, sl] + r2_v[b, t, sl]
                o_v[b, t, sl] = v
                return s + v, q + v * v

            zero = jnp.zeros((LANES,), jnp.float32)
            s, q = lax.fori_loop(0, NVREG, pass1, (zero, zero), unroll=8)
            mean = _lane_sum(s) * _INV_H
            var = _lane_sum(q) * _INV_H - mean * mean
            inv = _rsqrt(var + _EPS)

            def pass2(j, c):
                sl = pl.ds(j * LANES, LANES)
                sv = w_v[sl] * inv
                o_v[b, t, sl] = (o_v[b, t, sl] - mean) * sv + b_v[sl]
                return c

            lax.fori_loop(0, NVREG, pass2, 0, unroll=8)
            return tc

        lax.fori_loop(0, CHUNK, token, 0)

    fetch(0, 0)

    def pair(p, carry):
        for b in (0, 1):
            g = p * 2 + b
            wait_fetch(g, b)

            @pl.when(g + 1 < NGROUP)
            def _():
                fetch(g + 1, 1 - b)

            base = base0 + g * CHUNK

            @pl.when(g >= 2)
            def _():
                pltpu.make_async_copy(
                    o_v.at[b], out_hbm.at[pl.ds(base, CHUNK)], osem[b]).wait()

            compute(b)
            pltpu.async_copy(o_v.at[b], out_hbm.at[pl.ds(base, CHUNK)], osem[b])
        return carry

    lax.fori_loop(0, NGROUP // 2, pair, 0)
    for b in (0, 1):
        base = base0 + (NGROUP - 2 + b) * CHUNK
        pltpu.make_async_copy(
            o_v.at[b], out_hbm.at[pl.ds(base, CHUNK)], osem[b]).wait()


@jax.jit
def _fc_embed(x2d, ids, t1, t2, w, b):
    mesh = plsc.VectorSubcoreMesh(core_axis_name="c", subcore_axis_name="s")
    f = functools.partial(
        pl.kernel,
        mesh=mesh,
        out_type=jax.ShapeDtypeStruct((N_TOK, HIDDEN), jnp.float32),
        scratch_types=[
            pltpu.VMEM((2, CHUNK), jnp.int32),
            pltpu.VMEM((2, CHUNK, HIDDEN), jnp.float32),
            pltpu.VMEM((2, CHUNK, HIDDEN), jnp.float32),
            pltpu.VMEM((2, CHUNK, HIDDEN), jnp.float32),
            pltpu.VMEM((2, CHUNK, HIDDEN), jnp.float32),
            pltpu.VMEM((HIDDEN,), jnp.float32),
            pltpu.VMEM((HIDDEN,), jnp.float32),
            pltpu.SemaphoreType.DMA,
            pltpu.SemaphoreType.DMA,
            pltpu.SemaphoreType.DMA,
            pltpu.SemaphoreType.DMA,
        ],
    )(_body)
    return f(x2d, ids, t1, t2, w, b)


def kernel(inputs_embeds, position_ids, pos_table1, pos_table2, ln_weight, ln_bias):
    x2d = inputs_embeds.reshape(N_TOK, HIDDEN)
    ids = position_ids.reshape(N_TOK).astype(jnp.int32)
    out = _fc_embed(x2d, ids, pos_table1, pos_table2, ln_weight, ln_bias)
    return out.reshape(B, L, HIDDEN)


# speedup vs baseline: 1.4826x; 1.3519x over previous
import functools

import jax
import jax.numpy as jnp
from jax import lax
from jax.experimental import pallas as pl
from jax.experimental.pallas import tpu as pltpu
from jax.experimental.pallas import tpu_sc as plsc

MAX_POS = 8192
HIDDEN = 768
B = 4
L = 8192

N_TOK = B * L
LANES = 16
NC = 2
NS = 16
NW = NC * NS
TPW = N_TOK // NW
CHUNK = 16
NGROUP = TPW // CHUNK
NVREG = HIDDEN // LANES

_EPS = 1e-12
_INV_H = 1.0 / HIDDEN


def _rsqrt(u):
    yi = lax.bitcast_convert_type(u, jnp.int32)
    yi = jnp.full((LANES,), 0x5F3759DF, jnp.int32) - lax.shift_right_logical(
        yi, jnp.full((LANES,), 1, jnp.int32))
    g = lax.bitcast_convert_type(yi, jnp.float32)
    for _ in range(3):
        g = g * (1.5 - 0.5 * u * g * g)
    return g


_GATHER_DNUMS = lax.GatherDimensionNumbers(
    offset_dims=(), collapsed_slice_dims=(0,), start_index_map=(0,))


def _lane_sum(v):
    lanes = lax.iota(jnp.int32, LANES)
    for k in (8, 4, 2, 1):
        idx = lax.reshape(lanes ^ k, (LANES, 1))
        v = v + lax.gather(v, idx, _GATHER_DNUMS, (1,),
                           mode=lax.GatherScatterMode.PROMISE_IN_BOUNDS)
    return v


def _body(x_hbm, idx_hbm, t1_hbm, t2_hbm, w_hbm, b_hbm, out_hbm,
          idx_v, r1_v, r2_v, x_v, o_v, w_v, b_v,
          gsem0, gsem1, osem0, osem1):
    wid = lax.axis_index("s") * NC + lax.axis_index("c")
    base0 = wid * TPW
    gsem = (gsem0, gsem1)
    osem = (osem0, osem1)
    pltpu.sync_copy(w_hbm, w_v)
    pltpu.sync_copy(b_hbm, b_v)

    def fetch(g, b):
        base = base0 + g * CHUNK
        pltpu.sync_copy(idx_hbm.at[pl.ds(base, CHUNK)], idx_v.at[b])
        pltpu.async_copy(t1_hbm.at[idx_v.at[b]], r1_v.at[b], gsem[b])
        pltpu.async_copy(t2_hbm.at[idx_v.at[b]], r2_v.at[b], gsem[b])
        pltpu.async_copy(x_hbm.at[pl.ds(base, CHUNK)], x_v.at[b], gsem[b])

    def wait_fetch(g, b):
        base = base0 + g * CHUNK
        pltpu.make_async_copy(t1_hbm.at[idx_v.at[b]], r1_v.at[b], gsem[b]).wait()
        pltpu.make_async_copy(t2_hbm.at[idx_v.at[b]], r2_v.at[b], gsem[b]).wait()
        pltpu.make_async_copy(x_hbm.at[pl.ds(base, CHUNK)], x_v.at[b], gsem[b]).wait()

    def compute(b):
        zero = jnp.zeros((LANES,), jnp.float32)

        def token(t, tc):
            ss = [zero] * 4
            qq = [zero] * 4
            for j in range(NVREG):
                sl = pl.ds(j * LANES, LANES)
                v = r1_v[b, t, sl] * x_v[b, t, sl] + r2_v[b, t, sl]
                o_v[b, t, sl] = v
                a = j & 3
                ss[a] = ss[a] + v
                qq[a] = qq[a] + v * v
            s = (ss[0] + ss[1]) + (ss[2] + ss[3])
            q = (qq[0] + qq[1]) + (qq[2] + qq[3])
            mean = _lane_sum(s) * _INV_H
            var = _lane_sum(q) * _INV_H - mean * mean
            inv = _rsqrt(var + _EPS)
            for j in range(NVREG):
                sl = pl.ds(j * LANES, LANES)
                o_v[b, t, sl] = (o_v[b, t, sl] - mean) * (w_v[sl] * inv) + b_v[sl]
            return tc

        lax.fori_loop(0, CHUNK, token, 0)

    fetch(0, 0)

    def pair(p, carry):
        for b in (0, 1):
            g = p * 2 + b
            wait_fetch(g, b)

            @pl.when(g + 1 < NGROUP)
            def _():
                fetch(g + 1, 1 - b)

            base = base0 + g * CHUNK

            @pl.when(g >= 2)
            def _():
                pltpu.make_async_copy(
                    o_v.at[b], out_hbm.at[pl.ds(base, CHUNK)], osem[b]).wait()

            compute(b)
            pltpu.async_copy(o_v.at[b], out_hbm.at[pl.ds(base, CHUNK)], osem[b])
        return carry

    lax.fori_loop(0, NGROUP // 2, pair, 0)
    for b in (0, 1):
        base = base0 + (NGROUP - 2 + b) * CHUNK
        pltpu.make_async_copy(
            o_v.at[b], out_hbm.at[pl.ds(base, CHUNK)], osem[b]).wait()


@jax.jit
def _fc_embed(x2d, ids, t1, t2, w, b):
    mesh = plsc.VectorSubcoreMesh(core_axis_name="c", subcore_axis_name="s")
    f = functools.partial(
        pl.kernel,
        mesh=mesh,
        out_type=jax.ShapeDtypeStruct((N_TOK, HIDDEN), jnp.float32),
        scratch_types=[
            pltpu.VMEM((2, CHUNK), jnp.int32),
            pltpu.VMEM((2, CHUNK, HIDDEN), jnp.float32),
            pltpu.VMEM((2, CHUNK, HIDDEN), jnp.float32),
            pltpu.VMEM((2, CHUNK, HIDDEN), jnp.float32),
            pltpu.VMEM((2, CHUNK, HIDDEN), jnp.float32),
            pltpu.VMEM((HIDDEN,), jnp.float32),
            pltpu.VMEM((HIDDEN,), jnp.float32),
            pltpu.SemaphoreType.DMA,
            pltpu.SemaphoreType.DMA,
            pltpu.SemaphoreType.DMA,
            pltpu.SemaphoreType.DMA,
        ],
    )(_body)
    return f(x2d, ids, t1, t2, w, b)


def kernel(inputs_embeds, position_ids, pos_table1, pos_table2, ln_weight, ln_bias):
    x2d = inputs_embeds.reshape(N_TOK, HIDDEN)
    ids = position_ids.reshape(N_TOK).astype(jnp.int32)
    out = _fc_embed(x2d, ids, pos_table1, pos_table2, ln_weight, ln_bias)
    return out.reshape(B, L, HIDDEN)


# speedup vs baseline: 2.2901x; 1.5447x over previous
import functools

import jax
import jax.numpy as jnp
from jax import lax
from jax.experimental import pallas as pl
from jax.experimental.pallas import tpu as pltpu
from jax.experimental.pallas import tpu_sc as plsc

MAX_POS = 8192
HIDDEN = 768
B = 4
L = 8192

N_TOK = B * L
LANES = 16
NC = 2
NS = 16
NW = NC * NS
TPW = N_TOK // NW
CHUNK = 16
NGROUP = TPW // CHUNK
NVREG = HIDDEN // LANES

_EPS = 1e-12
_INV_H = 1.0 / HIDDEN


def _rsqrt(u):
    yi = lax.bitcast_convert_type(u, jnp.int32)
    yi = jnp.full((LANES,), 0x5F3759DF, jnp.int32) - lax.shift_right_logical(
        yi, jnp.full((LANES,), 1, jnp.int32))
    g = lax.bitcast_convert_type(yi, jnp.float32)
    for _ in range(3):
        g = g * (1.5 - 0.5 * u * g * g)
    return g


_GATHER_DNUMS = lax.GatherDimensionNumbers(
    offset_dims=(), collapsed_slice_dims=(0,), start_index_map=(0,))


def _lane_sum(v):
    lanes = lax.iota(jnp.int32, LANES)
    for k in (8, 4, 2, 1):
        idx = lax.reshape(lanes ^ k, (LANES, 1))
        v = v + lax.gather(v, idx, _GATHER_DNUMS, (1,),
                           mode=lax.GatherScatterMode.PROMISE_IN_BOUNDS)
    return v


def _body(x_hbm, idx_hbm, t1_hbm, t2_hbm, w_hbm, b_hbm, out_hbm,
          idx_v, r1_v, r2_v, x_v, o_v, w_v, b_v,
          gsem0, gsem1, osem0, osem1):
    wid = lax.axis_index("s") * NC + lax.axis_index("c")
    base0 = wid * TPW
    gsem = (gsem0, gsem1)
    osem = (osem0, osem1)
    pltpu.sync_copy(w_hbm, w_v)
    pltpu.sync_copy(b_hbm, b_v)

    def fetch(g, b):
        base = base0 + g * CHUNK
        pltpu.sync_copy(idx_hbm.at[pl.ds(base, CHUNK)], idx_v.at[b])
        pltpu.async_copy(t1_hbm.at[idx_v.at[b]], r1_v.at[b], gsem[b])
        pltpu.async_copy(t2_hbm.at[idx_v.at[b]], r2_v.at[b], gsem[b])
        pltpu.async_copy(x_hbm.at[pl.ds(base, CHUNK)], x_v.at[b], gsem[b])

    def wait_fetch(g, b):
        base = base0 + g * CHUNK
        pltpu.make_async_copy(t1_hbm.at[idx_v.at[b]], r1_v.at[b], gsem[b]).wait()
        pltpu.make_async_copy(t2_hbm.at[idx_v.at[b]], r2_v.at[b], gsem[b]).wait()
        pltpu.make_async_copy(x_hbm.at[pl.ds(base, CHUNK)], x_v.at[b], gsem[b]).wait()

    def compute(b):
        zero = jnp.zeros((LANES,), jnp.float32)

        @plsc.parallel_loop(0, CHUNK)
        def token(t):
            @plsc.parallel_loop(0, NVREG, step=2, unroll=4,
                                carry=(zero, zero, zero, zero))
            def pass1(j, c):
                s0, q0, s1, q1 = c
                sl0 = pl.ds(j * LANES, LANES)
                sl1 = pl.ds((j + 1) * LANES, LANES)
                va = r1_v[b, t, sl0] * x_v[b, t, sl0] + r2_v[b, t, sl0]
                vb = r1_v[b, t, sl1] * x_v[b, t, sl1] + r2_v[b, t, sl1]
                o_v[b, t, sl0] = va
                o_v[b, t, sl1] = vb
                return s0 + va, q0 + va * va, s1 + vb, q1 + vb * vb

            s0, q0, s1, q1 = pass1
            s = s0 + s1
            q = q0 + q1
            mean = _lane_sum(s) * _INV_H
            var = _lane_sum(q) * _INV_H - mean * mean
            inv = _rsqrt(var + _EPS)

            @plsc.parallel_loop(0, NVREG, unroll=8)
            def pass2(j):
                sl = pl.ds(j * LANES, LANES)
                o_v[b, t, sl] = (o_v[b, t, sl] - mean) * (w_v[sl] * inv) + b_v[sl]

    fetch(0, 0)

    def pair(p, carry):
        for b in (0, 1):
            g = p * 2 + b
            wait_fetch(g, b)

            @pl.when(g + 1 < NGROUP)
            def _():
                fetch(g + 1, 1 - b)

            base = base0 + g * CHUNK

            @pl.when(g >= 2)
            def _():
                pltpu.make_async_copy(
                    o_v.at[b], out_hbm.at[pl.ds(base, CHUNK)], osem[b]).wait()

            compute(b)
            pltpu.async_copy(o_v.at[b], out_hbm.at[pl.ds(base, CHUNK)], osem[b])
        return carry

    lax.fori_loop(0, NGROUP // 2, pair, 0)
    for b in (0, 1):
        base = base0 + (NGROUP - 2 + b) * CHUNK
        pltpu.make_async_copy(
            o_v.at[b], out_hbm.at[pl.ds(base, CHUNK)], osem[b]).wait()


@jax.jit
def _fc_embed(x2d, ids, t1, t2, w, b):
    mesh = plsc.VectorSubcoreMesh(core_axis_name="c", subcore_axis_name="s")
    f = functools.partial(
        pl.kernel,
        mesh=mesh,
        out_type=jax.ShapeDtypeStruct((N_TOK, HIDDEN), jnp.float32),
        scratch_types=[
            pltpu.VMEM((2, CHUNK), jnp.int32),
            pltpu.VMEM((2, CHUNK, HIDDEN), jnp.float32),
            pltpu.VMEM((2, CHUNK, HIDDEN), jnp.float32),
            pltpu.VMEM((2, CHUNK, HIDDEN), jnp.float32),
            pltpu.VMEM((2, CHUNK, HIDDEN), jnp.float32),
            pltpu.VMEM((HIDDEN,), jnp.float32),
            pltpu.VMEM((HIDDEN,), jnp.float32),
            pltpu.SemaphoreType.DMA,
            pltpu.SemaphoreType.DMA,
            pltpu.SemaphoreType.DMA,
            pltpu.SemaphoreType.DMA,
        ],
    )(_body)
    return f(x2d, ids, t1, t2, w, b)


def kernel(inputs_embeds, position_ids, pos_table1, pos_table2, ln_weight, ln_bias):
    x2d = inputs_embeds.reshape(N_TOK, HIDDEN)
    ids = position_ids.reshape(N_TOK).astype(jnp.int32)
    out = _fc_embed(x2d, ids, pos_table1, pos_table2, ln_weight, ln_bias)
    return out.reshape(B, L, HIDDEN)


# speedup vs baseline: 3.0530x; 1.3331x over previous
import functools

import jax
import jax.numpy as jnp
from jax import lax
from jax.experimental import pallas as pl
from jax.experimental.pallas import tpu as pltpu
from jax.experimental.pallas import tpu_sc as plsc

MAX_POS = 8192
HIDDEN = 768
B = 4
L = 8192

N_TOK = B * L
LANES = 16
NC = 2
NS = 16
NW = NC * NS
TPW = N_TOK // NW
CHUNK = 16
NGROUP = TPW // CHUNK
NVREG = HIDDEN // LANES

_EPS = 1e-12
_INV_H = 1.0 / HIDDEN


def _rsqrt(u):
    yi = lax.bitcast_convert_type(u, jnp.int32)
    yi = jnp.full((LANES,), 0x5F3759DF, jnp.int32) - lax.shift_right_logical(
        yi, jnp.full((LANES,), 1, jnp.int32))
    g = lax.bitcast_convert_type(yi, jnp.float32)
    for _ in range(3):
        g = g * (1.5 - 0.5 * u * g * g)
    return g


_GATHER_DNUMS = lax.GatherDimensionNumbers(
    offset_dims=(), collapsed_slice_dims=(0,), start_index_map=(0,))


def _lane_sum(v):
    lanes = lax.iota(jnp.int32, LANES)
    for k in (8, 4, 2, 1):
        idx = lax.reshape(lanes ^ k, (LANES, 1))
        v = v + lax.gather(v, idx, _GATHER_DNUMS, (1,),
                           mode=lax.GatherScatterMode.PROMISE_IN_BOUNDS)
    return v


def _body(x_hbm, idx_hbm, t1_hbm, t2_hbm, w_hbm, b_hbm, out_hbm,
          idx_v, r1_v, r2_v, x_v, o_v, w_v, b_v,
          gsem0, gsem1, osem0, osem1):
    wid = lax.axis_index("s") * NC + lax.axis_index("c")
    base0 = wid * TPW
    gsem = (gsem0, gsem1)
    osem = (osem0, osem1)
    pltpu.sync_copy(w_hbm, w_v)
    pltpu.sync_copy(b_hbm, b_v)

    def fetch(g, b):
        base = base0 + g * CHUNK
        pltpu.sync_copy(idx_hbm.at[pl.ds(base, CHUNK)], idx_v.at[b])
        pltpu.async_copy(t1_hbm.at[idx_v.at[b]], r1_v.at[b], gsem[b])
        pltpu.async_copy(t2_hbm.at[idx_v.at[b]], r2_v.at[b], gsem[b])
        pltpu.async_copy(x_hbm.at[pl.ds(base, CHUNK)], x_v.at[b], gsem[b])

    def wait_fetch(g, b):
        base = base0 + g * CHUNK
        pltpu.make_async_copy(t1_hbm.at[idx_v.at[b]], r1_v.at[b], gsem[b]).wait()
        pltpu.make_async_copy(t2_hbm.at[idx_v.at[b]], r2_v.at[b], gsem[b]).wait()
        pltpu.make_async_copy(x_hbm.at[pl.ds(base, CHUNK)], x_v.at[b], gsem[b]).wait()

    def compute(b):
        zero = jnp.zeros((LANES,), jnp.float32)

        @plsc.parallel_loop(0, CHUNK, step=2)
        def token(t):
            t1 = t + 1

            @plsc.parallel_loop(0, NVREG, step=2, unroll=2,
                                carry=(zero, zero, zero, zero,
                                       zero, zero, zero, zero))
            def pass1(j, c):
                sa0, qa0, sa1, qa1, sb0, qb0, sb1, qb1 = c
                sl0 = pl.ds(j * LANES, LANES)
                sl1 = pl.ds((j + 1) * LANES, LANES)
                va0 = r1_v[b, t, sl0] * x_v[b, t, sl0] + r2_v[b, t, sl0]
                va1 = r1_v[b, t, sl1] * x_v[b, t, sl1] + r2_v[b, t, sl1]
                vb0 = r1_v[b, t1, sl0] * x_v[b, t1, sl0] + r2_v[b, t1, sl0]
                vb1 = r1_v[b, t1, sl1] * x_v[b, t1, sl1] + r2_v[b, t1, sl1]
                o_v[b, t, sl0] = va0
                o_v[b, t, sl1] = va1
                o_v[b, t1, sl0] = vb0
                o_v[b, t1, sl1] = vb1
                return (sa0 + va0, qa0 + va0 * va0, sa1 + va1, qa1 + va1 * va1,
                        sb0 + vb0, qb0 + vb0 * vb0, sb1 + vb1, qb1 + vb1 * vb1)

            sa0, qa0, sa1, qa1, sb0, qb0, sb1, qb1 = pass1
            sa = _lane_sum(sa0 + sa1)
            qa = _lane_sum(qa0 + qa1)
            sb = _lane_sum(sb0 + sb1)
            qb = _lane_sum(qb0 + qb1)
            ma = sa * _INV_H
            mb = sb * _INV_H
            ia = _rsqrt(qa * _INV_H - ma * ma + _EPS)
            ib = _rsqrt(qb * _INV_H - mb * mb + _EPS)

            @plsc.parallel_loop(0, NVREG, unroll=4)
            def pass2(j):
                sl = pl.ds(j * LANES, LANES)
                wv = w_v[sl]
                bv = b_v[sl]
                o_v[b, t, sl] = (o_v[b, t, sl] - ma) * (wv * ia) + bv
                o_v[b, t1, sl] = (o_v[b, t1, sl] - mb) * (wv * ib) + bv

    fetch(0, 0)

    def pair(p, carry):
        for b in (0, 1):
            g = p * 2 + b
            wait_fetch(g, b)

            @pl.when(g + 1 < NGROUP)
            def _():
                fetch(g + 1, 1 - b)

            base = base0 + g * CHUNK

            @pl.when(g >= 2)
            def _():
                pltpu.make_async_copy(
                    o_v.at[b], out_hbm.at[pl.ds(base, CHUNK)], osem[b]).wait()

            compute(b)
            pltpu.async_copy(o_v.at[b], out_hbm.at[pl.ds(base, CHUNK)], osem[b])
        return carry

    lax.fori_loop(0, NGROUP // 2, pair, 0)
    for b in (0, 1):
        base = base0 + (NGROUP - 2 + b) * CHUNK
        pltpu.make_async_copy(
            o_v.at[b], out_hbm.at[pl.ds(base, CHUNK)], osem[b]).wait()


@jax.jit
def _fc_embed(x2d, ids, t1, t2, w, b):
    mesh = plsc.VectorSubcoreMesh(core_axis_name="c", subcore_axis_name="s")
    f = functools.partial(
        pl.kernel,
        mesh=mesh,
        out_type=jax.ShapeDtypeStruct((N_TOK, HIDDEN), jnp.float32),
        scratch_types=[
            pltpu.VMEM((2, CHUNK), jnp.int32),
            pltpu.VMEM((2, CHUNK, HIDDEN), jnp.float32),
            pltpu.VMEM((2, CHUNK, HIDDEN), jnp.float32),
            pltpu.VMEM((2, CHUNK, HIDDEN), jnp.float32),
            pltpu.VMEM((2, CHUNK, HIDDEN), jnp.float32),
            pltpu.VMEM((HIDDEN,), jnp.float32),
            pltpu.VMEM((HIDDEN,), jnp.float32),
            pltpu.SemaphoreType.DMA,
            pltpu.SemaphoreType.DMA,
            pltpu.SemaphoreType.DMA,
            pltpu.SemaphoreType.DMA,
        ],
    )(_body)
    return f(x2d, ids, t1, t2, w, b)


def kernel(inputs_embeds, position_ids, pos_table1, pos_table2, ln_weight, ln_bias):
    x2d = inputs_embeds.reshape(N_TOK, HIDDEN)
    ids = position_ids.reshape(N_TOK).astype(jnp.int32)
    out = _fc_embed(x2d, ids, pos_table1, pos_table2, ln_weight, ln_bias)
    return out.reshape(B, L, HIDDEN)


# speedup vs baseline: 3.9412x; 1.2909x over previous
import functools

import jax
import jax.numpy as jnp
from jax import lax
from jax.experimental import pallas as pl
from jax.experimental.pallas import tpu as pltpu
from jax.experimental.pallas import tpu_sc as plsc

MAX_POS = 8192
HIDDEN = 768
B = 4
L = 8192

N_TOK = B * L
LANES = 16
NC = 2
NS = 16
NW = NC * NS
TPW = N_TOK // NW
CHUNK = 16
NGROUP = TPW // CHUNK
NVREG = HIDDEN // LANES

_EPS = 1e-12
_INV_H = 1.0 / HIDDEN


def _rsqrt(u):
    yi = lax.bitcast_convert_type(u, jnp.int32)
    yi = jnp.full((LANES,), 0x5F3759DF, jnp.int32) - lax.shift_right_logical(
        yi, jnp.full((LANES,), 1, jnp.int32))
    g = lax.bitcast_convert_type(yi, jnp.float32)
    for _ in range(3):
        g = g * (1.5 - 0.5 * u * g * g)
    return g


_GATHER_DNUMS = lax.GatherDimensionNumbers(
    offset_dims=(), collapsed_slice_dims=(0,), start_index_map=(0,))


def _lane_sum(v):
    lanes = lax.iota(jnp.int32, LANES)
    for k in (8, 4, 2, 1):
        idx = lax.reshape(lanes ^ k, (LANES, 1))
        v = v + lax.gather(v, idx, _GATHER_DNUMS, (1,),
                           mode=lax.GatherScatterMode.PROMISE_IN_BOUNDS)
    return v


def _body(x_hbm, idx_hbm, t1_hbm, t2_hbm, w_hbm, b_hbm, out_hbm,
          idx_v, r1_v, r2_v, x_v, o_v, w_v, b_v,
          gsem0, gsem1, osem0, osem1):
    wid = lax.axis_index("s") * NC + lax.axis_index("c")
    base0 = wid * TPW
    gsem = (gsem0, gsem1)
    osem = (osem0, osem1)
    pltpu.sync_copy(w_hbm, w_v)
    pltpu.sync_copy(b_hbm, b_v)

    def fetch(g, b):
        base = base0 + g * CHUNK
        pltpu.sync_copy(idx_hbm.at[pl.ds(base, CHUNK)], idx_v.at[b])
        pltpu.async_copy(t1_hbm.at[idx_v.at[b]], r1_v.at[b], gsem[b])
        pltpu.async_copy(t2_hbm.at[idx_v.at[b]], r2_v.at[b], gsem[b])
        pltpu.async_copy(x_hbm.at[pl.ds(base, CHUNK)], x_v.at[b], gsem[b])

    def wait_fetch(g, b):
        base = base0 + g * CHUNK
        pltpu.make_async_copy(t1_hbm.at[idx_v.at[b]], r1_v.at[b], gsem[b]).wait()
        pltpu.make_async_copy(t2_hbm.at[idx_v.at[b]], r2_v.at[b], gsem[b]).wait()
        pltpu.make_async_copy(x_hbm.at[pl.ds(base, CHUNK)], x_v.at[b], gsem[b]).wait()

    def compute(b):
        zero = jnp.zeros((LANES,), jnp.float32)

        def pass1_pair(t0, t1):
            @plsc.parallel_loop(0, NVREG, step=2, unroll=2,
                                carry=(zero, zero, zero, zero,
                                       zero, zero, zero, zero))
            def p1(j, c):
                sa0, qa0, sa1, qa1, sb0, qb0, sb1, qb1 = c
                sl0 = pl.ds(j * LANES, LANES)
                sl1 = pl.ds((j + 1) * LANES, LANES)
                va0 = r1_v[b, t0, sl0] * x_v[b, t0, sl0] + r2_v[b, t0, sl0]
                va1 = r1_v[b, t0, sl1] * x_v[b, t0, sl1] + r2_v[b, t0, sl1]
                vb0 = r1_v[b, t1, sl0] * x_v[b, t1, sl0] + r2_v[b, t1, sl0]
                vb1 = r1_v[b, t1, sl1] * x_v[b, t1, sl1] + r2_v[b, t1, sl1]
                o_v[b, t0, sl0] = va0
                o_v[b, t0, sl1] = va1
                o_v[b, t1, sl0] = vb0
                o_v[b, t1, sl1] = vb1
                return (sa0 + va0, qa0 + va0 * va0, sa1 + va1, qa1 + va1 * va1,
                        sb0 + vb0, qb0 + vb0 * vb0, sb1 + vb1, qb1 + vb1 * vb1)

            sa0, qa0, sa1, qa1, sb0, qb0, sb1, qb1 = p1
            return sa0 + sa1, qa0 + qa1, sb0 + sb1, qb0 + qb1

        TGRP = 8

        @plsc.parallel_loop(0, CHUNK, step=TGRP)
        def token(t):
            means = []
            invs = []
            for k in range(0, TGRP, 2):
                sa, qa, sb, qb = pass1_pair(t + k, t + k + 1)
                for s, q in ((sa, qa), (sb, qb)):
                    m = _lane_sum(s) * _INV_H
                    means.append(m)
                    invs.append(_rsqrt(_lane_sum(q) * _INV_H - m * m + _EPS))

            @plsc.parallel_loop(0, NVREG, unroll=2)
            def pass2(j):
                sl = pl.ds(j * LANES, LANES)
                wv = w_v[sl]
                bv = b_v[sl]
                for k in range(TGRP):
                    o_v[b, t + k, sl] = (
                        (o_v[b, t + k, sl] - means[k]) * (wv * invs[k]) + bv)

    fetch(0, 0)

    def pair(p, carry):
        for b in (0, 1):
            g = p * 2 + b
            wait_fetch(g, b)

            @pl.when(g + 1 < NGROUP)
            def _():
                fetch(g + 1, 1 - b)

            base = base0 + g * CHUNK

            @pl.when(g >= 2)
            def _():
                pltpu.make_async_copy(
                    o_v.at[b], out_hbm.at[pl.ds(base, CHUNK)], osem[b]).wait()

            compute(b)
            pltpu.async_copy(o_v.at[b], out_hbm.at[pl.ds(base, CHUNK)], osem[b])
        return carry

    lax.fori_loop(0, NGROUP // 2, pair, 0)
    for b in (0, 1):
        base = base0 + (NGROUP - 2 + b) * CHUNK
        pltpu.make_async_copy(
            o_v.at[b], out_hbm.at[pl.ds(base, CHUNK)], osem[b]).wait()


@jax.jit
def _fc_embed(x2d, ids, t1, t2, w, b):
    mesh = plsc.VectorSubcoreMesh(core_axis_name="c", subcore_axis_name="s")
    f = functools.partial(
        pl.kernel,
        mesh=mesh,
        out_type=jax.ShapeDtypeStruct((N_TOK, HIDDEN), jnp.float32),
        scratch_types=[
            pltpu.VMEM((2, CHUNK), jnp.int32),
            pltpu.VMEM((2, CHUNK, HIDDEN), jnp.float32),
            pltpu.VMEM((2, CHUNK, HIDDEN), jnp.float32),
            pltpu.VMEM((2, CHUNK, HIDDEN), jnp.float32),
            pltpu.VMEM((2, CHUNK, HIDDEN), jnp.float32),
            pltpu.VMEM((HIDDEN,), jnp.float32),
            pltpu.VMEM((HIDDEN,), jnp.float32),
            pltpu.SemaphoreType.DMA,
            pltpu.SemaphoreType.DMA,
            pltpu.SemaphoreType.DMA,
            pltpu.SemaphoreType.DMA,
        ],
    )(_body)
    return f(x2d, ids, t1, t2, w, b)


def kernel(inputs_embeds, position_ids, pos_table1, pos_table2, ln_weight, ln_bias):
    x2d = inputs_embeds.reshape(N_TOK, HIDDEN)
    ids = position_ids.reshape(N_TOK).astype(jnp.int32)
    out = _fc_embed(x2d, ids, pos_table1, pos_table2, ln_weight, ln_bias)
    return out.reshape(B, L, HIDDEN)
